# baseline (speedup 1.0000x reference)
import functools

import jax
import jax.numpy as jnp
from jax import lax
from jax.experimental import pallas as pl
from jax.experimental.pallas import tpu as pltpu
from jax.experimental.pallas import tpu_sc as plsc

NC = 2
NS = 16
NW = NC * NS
LANES = 16


def _sc_detile_table(emb_t, tail_flat, V, D):
    TCOLS = V // 128
    per_w = TCOLS // NW
    CH = 4
    nch = per_w // CH
    CW = CH * 128
    vpw = per_w * 128
    E0 = NW * vpw
    extra_full = TCOLS - NW * per_w
    rem = V - TCOLS * 128
    assert nch >= 3 and nch % 2 == 1
    mesh = plsc.VectorSubcoreMesh(
        core_axis_name="c", subcore_axis_name="s",
        num_cores=NC, num_subcores=NS)

    @functools.partial(
        pl.kernel,
        out_type=jax.ShapeDtypeStruct((V * D,), jnp.float32),
        mesh=mesh,
        compiler_params=pltpu.CompilerParams(needs_layout_passes=False),
        scratch_types=[
            pltpu.VMEM((D, CW), jnp.float32),
            pltpu.VMEM((D, CW), jnp.float32),
            pltpu.VMEM((CW * D,), jnp.float32),
            pltpu.VMEM((CW * D,), jnp.float32),
            pltpu.SemaphoreType.DMA,
            pltpu.SemaphoreType.DMA,
            pltpu.SemaphoreType.DMA,
            pltpu.SemaphoreType.DMA,
        ],
    )
    def body(t_hbm, tail_hbm, out_hbm, ib0, ib1, ob0, ob1, si0, si1, so0, so1):
        wid = lax.axis_index("s") * NC + lax.axis_index("c")
        vb = pl.multiple_of(wid * vpw, 128)
        ibufs = (ib0, ib1)
        obufs = (ob0, ob1)
        sis = (si0, si1)
        sos = (so0, so1)
        d_lo = jnp.arange(LANES, dtype=jnp.int32)
        d_hi = d_lo + LANES

        def fire_in(c, b):
            start = pl.multiple_of(vb + c * CW, 128)
            pltpu.make_async_copy(
                t_hbm.at[:, pl.ds(start, CW)], ibufs[b], sis[b]).start()

        def drain_in(b):
            pltpu.make_async_copy(
                t_hbm.at[:, pl.ds(vb, CW)], ibufs[b], sis[b]).wait()

        def fire_out(c, b):
            pltpu.make_async_copy(
                obufs[b], out_hbm.at[pl.ds((vb + c * CW) * D, CW * D)],
                sos[b]).start()

        def drain_out(b):
            pltpu.make_async_copy(
                obufs[b], out_hbm.at[pl.ds(vb * D, CW * D)], sos[b]).wait()

        def transpose_chunk(b):
            ib = ibufs[b]
            ob = obufs[b]

            def tbody(v4, _):
                for u in range(4):
                    v = v4 * 4 + u
                    vv = jnp.full((LANES,), 0, jnp.int32) + v
                    lo = plsc.load_gather(ib, [d_lo, vv])
                    hi = plsc.load_gather(ib, [d_hi, vv])
                    ob[pl.ds(v * D, LANES)] = lo
                    ob[pl.ds(v * D + LANES, LANES)] = hi
                return 0

            lax.fori_loop(0, CW // 4, tbody, 0)

        def process(c, b, first):
            drain_in(b)
            if not first:
                drain_out(b)
            transpose_chunk(b)
            fire_out(c, b)
            fire_in(jnp.minimum(c + 2, nch - 1), b)

        fire_in(0, 0)
        fire_in(1, 1)
        process(0, 0, True)
        process(1, 1, True)

        def pbody(p, _):
            process(2 * p, 0, False)
            process(2 * p + 1, 1, False)
            return 0

        lax.fori_loop(1, (nch - 1) // 2, pbody, 0)
        process(nch - 1, 0, False)
        drain_in(0)
        drain_in(1)
        drain_out(0)
        drain_out(1)

        @pl.when(wid < extra_full)
        def _():
            v0 = pl.multiple_of(E0 + wid * 128, 128)
            pltpu.sync_copy(t_hbm.at[:, pl.ds(v0, 128)],
                            ib0.at[:, pl.ds(0, 128)])

            def ebody(v, _):
                vv = jnp.full((LANES,), 0, jnp.int32) + v
                lo = plsc.load_gather(ib0, [d_lo, vv])
                hi = plsc.load_gather(ib0, [d_hi, vv])
                ob0[pl.ds(v * D, LANES)] = lo
                ob0[pl.ds(v * D + LANES, LANES)] = hi
                return 0

            lax.fori_loop(0, 128, ebody, 0)
            pltpu.sync_copy(ob0.at[pl.ds(0, 128 * D)],
                            out_hbm.at[pl.ds(v0 * D, 128 * D)])

        if rem:
            @pl.when(wid == extra_full)
            def _():
                v0 = E0 + extra_full * 128
                pltpu.sync_copy(tail_hbm, ob0.at[pl.ds(0, rem * D)])
                pltpu.sync_copy(ob0.at[pl.ds(0, rem * D)],
                                out_hbm.at[pl.ds(v0 * D, rem * D)])

    return body(emb_t, tail_flat)


def _sc_sum_pool(x_r, emb_table, B, H, D, spw, chunk):
    h2 = H // 2
    nchunks = spw // chunk
    mesh = plsc.VectorSubcoreMesh(
        core_axis_name="c", subcore_axis_name="s",
        num_cores=NC, num_subcores=NS)

    @functools.partial(
        pl.kernel,
        out_type=jax.ShapeDtypeStruct((B, D), jnp.float32),
        mesh=mesh,
        compiler_params=pltpu.CompilerParams(use_tc_tiling_on_sc=False),
        scratch_types=[
            pltpu.VMEM((chunk, 2, h2), jnp.int32),
            pltpu.VMEM((chunk, 2, h2), jnp.int32),
            pltpu.VMEM((2, h2, D), jnp.float32),
            pltpu.VMEM((2, h2, D), jnp.float32),
            pltpu.VMEM((spw, D), jnp.float32),
            pltpu.SemaphoreType.DMA,
            pltpu.SemaphoreType.DMA,
            pltpu.SemaphoreType.DMA,
        ],
    )
    def body(x_hbm, emb_hbm, out_hbm, ibuf0, ibuf1, rbuf0, rbuf1,
             pooled, semi, sem0, sem1):
        wid = lax.axis_index("s") * NC + lax.axis_index("c")
        base = wid * spw
        ibufs = (ibuf0, ibuf1)
        rbufs = (rbuf0, rbuf1)
        sems = (sem0, sem1)

        def idx_copy(c):
            pltpu.make_async_copy(
                x_hbm.at[pl.ds(base + c * chunk, chunk)],
                ibufs[c % 2], semi).start()

        def idx_wait(c):
            pltpu.make_async_copy(
                x_hbm.at[pl.ds(base + c * chunk, chunk)],
                ibufs[c % 2], semi).wait()

        def fire(ib, i, rbi):
            for j in range(2):
                pltpu.make_async_copy(
                    emb_hbm.at[ibufs[ib].at[i, j]],
                    rbufs[rbi].at[j], sems[rbi]).start()

        def drain(ib, i, rbi):
            for j in range(2):
                pltpu.make_async_copy(
                    emb_hbm.at[ibufs[ib].at[i, j]],
                    rbufs[rbi].at[j], sems[rbi]).wait()

        def reduce(rbi, sl):
            rb = rbufs[rbi]
            zero = jnp.zeros((LANES,), jnp.float32)

            def rbody(r2, accs):
                a0, a1, a2, a3, a4, a5, a6, a7 = accs
                r = 2 * r2
                a0 = a0 + rb[0, r, pl.ds(0, LANES)]
                a1 = a1 + rb[0, r, pl.ds(LANES, LANES)]
                a2 = a2 + rb[1, r, pl.ds(0, LANES)]
                a3 = a3 + rb[1, r, pl.ds(LANES, LANES)]
                a4 = a4 + rb[0, r + 1, pl.ds(0, LANES)]
                a5 = a5 + rb[0, r + 1, pl.ds(LANES, LANES)]
                a6 = a6 + rb[1, r + 1, pl.ds(0, LANES)]
                a7 = a7 + rb[1, r + 1, pl.ds(LANES, LANES)]
                return (a0, a1, a2, a3, a4, a5, a6, a7)

            a = lax.fori_loop(0, h2 // 2, rbody, (zero,) * 8)
            lo = (a[0] + a[2]) + (a[4] + a[6])
            hi = (a[1] + a[3]) + (a[5] + a[7])
            pooled[sl, pl.ds(0, LANES)] = lo
            pooled[sl, pl.ds(LANES, LANES)] = hi

        idx_copy(0)
        idx_wait(0)
        for c in range(nchunks):
            ib = c % 2
            if c + 1 < nchunks:
                idx_copy(c + 1)
            fire(ib, 0, 0)
            fire(ib, 1, 1)

            def pbody(p, _, ib=ib, c=c):
                i0 = 2 * p
                last = chunk - 1
                drain(ib, i0, 0)
                reduce(0, c * chunk + i0)
                fire(ib, jnp.minimum(i0 + 2, last), 0)
                drain(ib, i0 + 1, 1)
                reduce(1, c * chunk + i0 + 1)
                fire(ib, jnp.minimum(i0 + 3, last), 1)
                return 0

            lax.fori_loop(0, chunk // 2, pbody, 0)
            drain(ib, chunk - 1, 0)
            drain(ib, chunk - 1, 1)
            if c + 1 < nchunks:
                idx_wait(c + 1)

        pltpu.sync_copy(pooled, out_hbm.at[pl.ds(base, spw)])

    return body(x_r, emb_table)


def _tc_linear_relu(sums, fc_w, fc_b2, inv_h, B, D):
    nblk = 8
    blk = B // nblk

    def body(s_ref, w_ref, b_ref, o_ref):
        pooled = s_ref[...] * inv_h
        acc = lax.dot_general(
            pooled, w_ref[...], (((1,), (1,)), ((), ())),
            preferred_element_type=jnp.float32)
        o_ref[...] = jnp.maximum(acc + b_ref[...], 0.0)

    return pl.pallas_call(
        body,
        out_shape=jax.ShapeDtypeStruct((B, D), jnp.float32),
        grid=(nblk,),
        in_specs=[
            pl.BlockSpec((blk, D), lambda i: (i, 0)),
            pl.BlockSpec((D, D), lambda i: (0, 0)),
            pl.BlockSpec((1, D), lambda i: (0, 0)),
        ],
        out_specs=pl.BlockSpec((blk, D), lambda i: (i, 0)),
    )(sums, fc_w, fc_b2)


def kernel(x, emb_table, fc_w, fc_b):
    B, H = x.shape
    D = emb_table.shape[1]
    assert B % NW == 0 and H % 2 == 0 and H // 2 <= 128 and D == 2 * LANES
    spw = B // NW
    chunk = 128
    assert spw % chunk == 0 and chunk % 2 == 0

    x_r = x.astype(jnp.int32).reshape(B, 2, H // 2)
    V = emb_table.shape[0]
    n_tail = V % 128
    tail_flat = emb_table[V - n_tail:].reshape(-1)
    lin_flat = _sc_detile_table(emb_table.T, tail_flat, V, D)
    lin_table = lin_flat.reshape(V, D)
    sums = _sc_sum_pool(x_r, lin_table, B, H, D, spw, chunk)
    return _tc_linear_relu(sums, fc_w, fc_b.reshape(1, D), 1.0 / H, B, D)


# speedup vs baseline: 1.2018x; 1.2018x over previous
import functools

import jax
import jax.numpy as jnp
from jax import lax
from jax.experimental import pallas as pl
from jax.experimental.pallas import tpu as pltpu
from jax.experimental.pallas import tpu_sc as plsc

NC = 2
NS = 16
NW = NC * NS
LANES = 16


def _sc_detile_table(emb_t, tail_flat, V, D):
    TCOLS = V // 128
    per_w = TCOLS // NW
    CH = 4
    nch = per_w // CH
    CW = CH * 128
    vpw = per_w * 128
    E0 = NW * vpw
    extra_full = TCOLS - NW * per_w
    rem = V - TCOLS * 128
    assert nch >= 3 and nch % 2 == 1
    mesh = plsc.VectorSubcoreMesh(
        core_axis_name="c", subcore_axis_name="s",
        num_cores=NC, num_subcores=NS)

    @functools.partial(
        pl.kernel,
        out_type=jax.ShapeDtypeStruct((V * D,), jnp.float32),
        mesh=mesh,
        compiler_params=pltpu.CompilerParams(needs_layout_passes=False),
        scratch_types=[
            pltpu.VMEM((D, CW), jnp.float32),
            pltpu.VMEM((D, CW), jnp.float32),
            pltpu.VMEM((CW * D,), jnp.float32),
            pltpu.VMEM((CW * D,), jnp.float32),
            pltpu.SemaphoreType.DMA,
            pltpu.SemaphoreType.DMA,
            pltpu.SemaphoreType.DMA,
            pltpu.SemaphoreType.DMA,
        ],
    )
    def body(t_hbm, tail_hbm, out_hbm, ib0, ib1, ob0, ob1, si0, si1, so0, so1):
        wid = lax.axis_index("s") * NC + lax.axis_index("c")
        vb = pl.multiple_of(wid * vpw, 128)
        ibufs = (ib0, ib1)
        obufs = (ob0, ob1)
        sis = (si0, si1)
        sos = (so0, so1)
        d_lo = jnp.arange(LANES, dtype=jnp.int32)
        d_hi = d_lo + LANES

        def fire_in(c, b):
            start = pl.multiple_of(vb + c * CW, 128)
            pltpu.make_async_copy(
                t_hbm.at[:, pl.ds(start, CW)], ibufs[b], sis[b]).start()

        def drain_in(b):
            pltpu.make_async_copy(
                t_hbm.at[:, pl.ds(vb, CW)], ibufs[b], sis[b]).wait()

        def fire_out(c, b):
            pltpu.make_async_copy(
                obufs[b], out_hbm.at[pl.ds((vb + c * CW) * D, CW * D)],
                sos[b]).start()

        def drain_out(b):
            pltpu.make_async_copy(
                obufs[b], out_hbm.at[pl.ds(vb * D, CW * D)], sos[b]).wait()

        def transpose_chunk(b):
            ib = ibufs[b]
            ob = obufs[b]

            @plsc.parallel_loop(0, CW, step=8, unroll=2)
            def _(v8):
                vals = []
                for u in range(8):
                    vv = jnp.full((LANES,), 0, jnp.int32) + (v8 + u)
                    vals.append((plsc.load_gather(ib, [d_lo, vv]),
                                 plsc.load_gather(ib, [d_hi, vv])))
                for u, (lo, hi) in enumerate(vals):
                    ob[pl.ds((v8 + u) * D, LANES)] = lo
                    ob[pl.ds((v8 + u) * D + LANES, LANES)] = hi

        def process(c, b, first):
            drain_in(b)
            if not first:
                drain_out(b)
            transpose_chunk(b)
            fire_out(c, b)
            fire_in(jnp.minimum(c + 2, nch - 1), b)

        fire_in(0, 0)
        fire_in(1, 1)
        process(0, 0, True)
        process(1, 1, True)

        def pbody(p, _):
            process(2 * p, 0, False)
            process(2 * p + 1, 1, False)
            return 0

        lax.fori_loop(1, (nch - 1) // 2, pbody, 0)
        process(nch - 1, 0, False)
        drain_in(0)
        drain_in(1)
        drain_out(0)
        drain_out(1)

        @pl.when(wid < extra_full)
        def _():
            v0 = pl.multiple_of(E0 + wid * 128, 128)
            pltpu.sync_copy(t_hbm.at[:, pl.ds(v0, 128)],
                            ib0.at[:, pl.ds(0, 128)])

            def ebody(v, _):
                vv = jnp.full((LANES,), 0, jnp.int32) + v
                lo = plsc.load_gather(ib0, [d_lo, vv])
                hi = plsc.load_gather(ib0, [d_hi, vv])
                ob0[pl.ds(v * D, LANES)] = lo
                ob0[pl.ds(v * D + LANES, LANES)] = hi
                return 0

            lax.fori_loop(0, 128, ebody, 0)
            pltpu.sync_copy(ob0.at[pl.ds(0, 128 * D)],
                            out_hbm.at[pl.ds(v0 * D, 128 * D)])

        if rem:
            @pl.when(wid == extra_full)
            def _():
                v0 = E0 + extra_full * 128
                pltpu.sync_copy(tail_hbm, ob0.at[pl.ds(0, rem * D)])
                pltpu.sync_copy(ob0.at[pl.ds(0, rem * D)],
                                out_hbm.at[pl.ds(v0 * D, rem * D)])

    return body(emb_t, tail_flat)


def _sc_sum_pool(x_r, emb_table, B, H, D, spw, chunk):
    h2 = H // 2
    nchunks = spw // chunk
    mesh = plsc.VectorSubcoreMesh(
        core_axis_name="c", subcore_axis_name="s",
        num_cores=NC, num_subcores=NS)

    @functools.partial(
        pl.kernel,
        out_type=jax.ShapeDtypeStruct((B, D), jnp.float32),
        mesh=mesh,
        compiler_params=pltpu.CompilerParams(use_tc_tiling_on_sc=False),
        scratch_types=[
            pltpu.VMEM((chunk, 2, h2), jnp.int32),
            pltpu.VMEM((chunk, 2, h2), jnp.int32),
            pltpu.VMEM((2, h2, D), jnp.float32),
            pltpu.VMEM((2, h2, D), jnp.float32),
            pltpu.VMEM((spw, D), jnp.float32),
            pltpu.SemaphoreType.DMA,
            pltpu.SemaphoreType.DMA,
            pltpu.SemaphoreType.DMA,
        ],
    )
    def body(x_hbm, emb_hbm, out_hbm, ibuf0, ibuf1, rbuf0, rbuf1,
             pooled, semi, sem0, sem1):
        wid = lax.axis_index("s") * NC + lax.axis_index("c")
        base = wid * spw
        ibufs = (ibuf0, ibuf1)
        rbufs = (rbuf0, rbuf1)
        sems = (sem0, sem1)

        def idx_copy(c):
            pltpu.make_async_copy(
                x_hbm.at[pl.ds(base + c * chunk, chunk)],
                ibufs[c % 2], semi).start()

        def idx_wait(c):
            pltpu.make_async_copy(
                x_hbm.at[pl.ds(base + c * chunk, chunk)],
                ibufs[c % 2], semi).wait()

        def fire(ib, i, rbi):
            for j in range(2):
                pltpu.make_async_copy(
                    emb_hbm.at[ibufs[ib].at[i, j]],
                    rbufs[rbi].at[j], sems[rbi]).start()

        def drain(ib, i, rbi):
            for j in range(2):
                pltpu.make_async_copy(
                    emb_hbm.at[ibufs[ib].at[i, j]],
                    rbufs[rbi].at[j], sems[rbi]).wait()

        def reduce(rbi, sl):
            rb = rbufs[rbi]
            zero = jnp.zeros((LANES,), jnp.float32)

            def rbody(r2, accs):
                a0, a1, a2, a3, a4, a5, a6, a7 = accs
                r = 2 * r2
                a0 = a0 + rb[0, r, pl.ds(0, LANES)]
                a1 = a1 + rb[0, r, pl.ds(LANES, LANES)]
                a2 = a2 + rb[1, r, pl.ds(0, LANES)]
                a3 = a3 + rb[1, r, pl.ds(LANES, LANES)]
                a4 = a4 + rb[0, r + 1, pl.ds(0, LANES)]
                a5 = a5 + rb[0, r + 1, pl.ds(LANES, LANES)]
                a6 = a6 + rb[1, r + 1, pl.ds(0, LANES)]
                a7 = a7 + rb[1, r + 1, pl.ds(LANES, LANES)]
                return (a0, a1, a2, a3, a4, a5, a6, a7)

            a = lax.fori_loop(0, h2 // 2, rbody, (zero,) * 8)
            lo = (a[0] + a[2]) + (a[4] + a[6])
            hi = (a[1] + a[3]) + (a[5] + a[7])
            pooled[sl, pl.ds(0, LANES)] = lo
            pooled[sl, pl.ds(LANES, LANES)] = hi

        idx_copy(0)
        idx_wait(0)
        for c in range(nchunks):
            ib = c % 2
            if c + 1 < nchunks:
                idx_copy(c + 1)
            fire(ib, 0, 0)
            fire(ib, 1, 1)

            def pbody(p, _, ib=ib, c=c):
                i0 = 2 * p
                last = chunk - 1
                drain(ib, i0, 0)
                reduce(0, c * chunk + i0)
                fire(ib, jnp.minimum(i0 + 2, last), 0)
                drain(ib, i0 + 1, 1)
                reduce(1, c * chunk + i0 + 1)
                fire(ib, jnp.minimum(i0 + 3, last), 1)
                return 0

            lax.fori_loop(0, chunk // 2, pbody, 0)
            drain(ib, chunk - 1, 0)
            drain(ib, chunk - 1, 1)
            if c + 1 < nchunks:
                idx_wait(c + 1)

        pltpu.sync_copy(pooled, out_hbm.at[pl.ds(base, spw)])

    return body(x_r, emb_table)


def _tc_linear_relu(sums, fc_w, fc_b2, inv_h, B, D):
    nblk = 8
    blk = B // nblk

    def body(s_ref, w_ref, b_ref, o_ref):
        pooled = s_ref[...] * inv_h
        acc = lax.dot_general(
            pooled, w_ref[...], (((1,), (1,)), ((), ())),
            preferred_element_type=jnp.float32)
        o_ref[...] = jnp.maximum(acc + b_ref[...], 0.0)

    return pl.pallas_call(
        body,
        out_shape=jax.ShapeDtypeStruct((B, D), jnp.float32),
        grid=(nblk,),
        in_specs=[
            pl.BlockSpec((blk, D), lambda i: (i, 0)),
            pl.BlockSpec((D, D), lambda i: (0, 0)),
            pl.BlockSpec((1, D), lambda i: (0, 0)),
        ],
        out_specs=pl.BlockSpec((blk, D), lambda i: (i, 0)),
    )(sums, fc_w, fc_b2)


def kernel(x, emb_table, fc_w, fc_b):
    B, H = x.shape
    D = emb_table.shape[1]
    assert B % NW == 0 and H % 2 == 0 and H // 2 <= 128 and D == 2 * LANES
    spw = B // NW
    chunk = 128
    assert spw % chunk == 0 and chunk % 2 == 0

    x_r = x.astype(jnp.int32).reshape(B, 2, H // 2)
    V = emb_table.shape[0]
    n_tail = V % 128
    tail_flat = emb_table[V - n_tail:].reshape(-1)
    lin_flat = _sc_detile_table(emb_table.T, tail_flat, V, D)
    lin_table = lin_flat.reshape(V, D)
    sums = _sc_sum_pool(x_r, lin_table, B, H, D, spw, chunk)
    return _tc_linear_relu(sums, fc_w, fc_b.reshape(1, D), 1.0 / H, B, D)
